# Initial kernel scaffold; baseline (speedup 1.0000x reference)
import jax
import jax.numpy as jnp
from jax.experimental import pallas as pl


def kernel(input, tables):
    raise NotImplementedError("write your pallas kernel here")


# speedup vs baseline: 1.0329x; 1.0329x over previous
import functools

import jax
import jax.numpy as jnp
from jax import lax
from jax.experimental import pallas as pl
from jax.experimental.pallas import tpu as pltpu
from jax.experimental.pallas import tpu_sc as plsc

F = 26
V = 100000
D = 64
B = 4096
L = 16

NC = 2
NS = 16
NW = NC * NS
BPW = B // NW
S = 32
NCHUNK = BPW // S
DV = D // L


def _body(fidx_hbm, tflat_hbm, out_hbm, idx_v, buf_v, acc_v, sem):
    wid = lax.axis_index("s") * NC + lax.axis_index("c")
    base = wid * BPW

    pltpu.sync_copy(fidx_hbm.at[wid], idx_v)

    for c in range(NCHUNK):
        copies = []
        for f in range(F):
            cp = pltpu.async_copy(
                tflat_hbm.at[idx_v.at[f, pl.ds(c * S, S)]],
                buf_v.at[f],
                sem,
            )
            copies.append(cp)
        for cp in copies:
            cp.wait()

        def acc_row(j, _):
            for d in range(DV):
                sl = pl.ds(d * L, L)
                v = buf_v[0, j, sl]
                for f in range(1, F):
                    v = v + buf_v[f, j, sl]
                acc_v[c * S + j, sl] = v
            return 0

        lax.fori_loop(0, S, acc_row, 0)

    pltpu.sync_copy(acc_v, out_hbm.at[pl.ds(base, BPW)])


@jax.jit
def _embed_sum(fidx, tflat):
    mesh = plsc.VectorSubcoreMesh(core_axis_name="c", subcore_axis_name="s")
    return pl.kernel(
        _body,
        out_type=jax.ShapeDtypeStruct((B, D), jnp.float32),
        mesh=mesh,
        scratch_types=[
            pltpu.VMEM((F, BPW), jnp.int32),
            pltpu.VMEM((F, S, D), jnp.float32),
            pltpu.VMEM((BPW, D), jnp.float32),
            pltpu.SemaphoreType.DMA,
        ],
        compiler_params=pltpu.CompilerParams(use_tc_tiling_on_sc=False),
    )(fidx, tflat)


def kernel(input, tables):
    fidx = input + jnp.arange(F, dtype=jnp.int32)[None, :] * V
    fidx = fidx.reshape(NW, BPW, F).transpose(0, 2, 1)
    tflat = tables.reshape(F * V, D)
    return _embed_sum(fidx, tflat)
